# baseline (speedup 1.0000x reference)
import jax
import jax.numpy as jnp
from jax.experimental import pallas as pl
from jax.experimental.pallas import tpu as pltpu

N = 10000
E = 160000
B = 1
S = 8
C = 1
H = 64
FF = 128
EMB = 16
K = 2


def _supports(edge_index, edge_weight):
    src, dst = edge_index[0], edge_index[1]
    deg_out = jax.ops.segment_sum(edge_weight, src, num_segments=N)
    deg_in = jax.ops.segment_sum(edge_weight, dst, num_segments=N)
    wf = edge_weight / jnp.maximum(deg_out[src], 1e-8)
    wb = edge_weight / jnp.maximum(deg_in[dst], 1e-8)
    return src, dst, wf, wb


def _prop(x, gi, si, w):
    msg = x[:, gi, :] * w[None, :, None]
    return jax.vmap(lambda m: jax.ops.segment_sum(m, si, num_segments=N))(msg)


def _diff_conv(x, p, sup, order):
    src, dst, wf, wb = sup
    out = x @ p['W0'] + p['b']
    xf, xb = x, x
    for k in range(order):
        xf = _prop(xf, dst, src, wf)
        xb = _prop(xb, src, dst, wb)
        out = out + xf @ p['Wf'][k] + xb @ p['Wb'][k]
    return out


def _dcrnn_cell(inp, h, p, sup):
    xh = jnp.concatenate([inp, h], axis=-1)
    r = jax.nn.sigmoid(_diff_conv(xh, p['r'], sup, K))
    u = jax.nn.sigmoid(_diff_conv(xh, p['u'], sup, K))
    xrh = jnp.concatenate([inp, r * h], axis=-1)
    c = jnp.tanh(_diff_conv(xrh, p['c'], sup, K))
    return u * h + (1.0 - u) * c


def _gril(x, mask, gp, sup):
    h0 = jnp.broadcast_to(gp['h0'][None], (x.shape[0], N, H))

    def step(h, xm):
        x_t, m_t = xm
        src, dst, wf, wb = sup
        mb = m_t > 0.5
        xs_hat_1 = h @ gp['first_stage']['W'] + gp['first_stage']['b']
        x1 = jnp.where(mb, x_t, xs_hat_1)
        dec_in = jnp.concatenate([x1, m_t], axis=-1)
        dp = gp['dec']
        z = jnp.concatenate([dec_in, h], axis=-1) @ dp['lin_in']['W'] + dp['lin_in']['b']
        gc = _prop(z, dst, src, wf) @ dp['gc']['Wf'] + _prop(z, src, dst, wb) @ dp['gc']['Wb'] + dp['gc']['b']
        o = jnp.concatenate([gc, h], axis=-1) @ dp['lin_out']['W'] + dp['lin_out']['b']
        o = jnp.where(o >= 0, o, dp['prelu_a'] * o)
        rep = jnp.concatenate([o, h], axis=-1)
        xs_hat_2 = rep @ dp['read_out']['W'] + dp['read_out']['b']
        x2 = jnp.where(mb, x_t, xs_hat_2)
        cell_in = jnp.concatenate([x2, m_t], axis=-1)
        h_new = _dcrnn_cell(cell_in, h, gp['cell'], sup)
        return h_new, (xs_hat_2, xs_hat_1, rep)

    xs = jnp.moveaxis(x, 1, 0)
    ms = jnp.moveaxis(mask, 1, 0)
    _, (imp, pred, rep) = jax.lax.scan(step, h0, (xs, ms))
    return jnp.moveaxis(imp, 0, 1), jnp.moveaxis(pred, 0, 1), jnp.moveaxis(rep, 0, 1)


_ROWS_BLK = 1000


def _readout_body(feat_ref, w1_ref, b1_ref, w2_ref, b2_ref, out_ref):
    f = feat_ref[...]
    hid = jnp.maximum(f @ w1_ref[...] + b1_ref[...], 0.0)
    out_ref[...] = hid @ w2_ref[...] + b2_ref[...]


def _readout(feat_pad, w1_pad, b1, w2, b2):
    rows = feat_pad.shape[0]
    fpad = feat_pad.shape[1]
    grid = (rows // _ROWS_BLK,)
    return pl.pallas_call(
        _readout_body,
        grid=grid,
        in_specs=[
            pl.BlockSpec((_ROWS_BLK, fpad), lambda i: (i, 0)),
            pl.BlockSpec((fpad, FF), lambda i: (0, 0)),
            pl.BlockSpec((FF,), lambda i: (0,)),
            pl.BlockSpec((FF, 128), lambda i: (0, 0)),
            pl.BlockSpec((128,), lambda i: (0,)),
        ],
        out_specs=pl.BlockSpec((_ROWS_BLK, 128), lambda i: (i, 0)),
        out_shape=jax.ShapeDtypeStruct((rows, 128), jnp.float32),
    )(feat_pad, w1_pad, b1, w2, b2)


def kernel(x, edge_index, edge_weight, mask, params):
    sup = _supports(edge_index, edge_weight)
    fwd_out, fwd_pred, fwd_repr = _gril(x, mask, params['fwd'], sup)
    b_out, b_pred, b_repr = _gril(x[:, ::-1], mask[:, ::-1], params['bwd'], sup)
    bwd_out, bwd_pred, bwd_repr = b_out[:, ::-1], b_pred[:, ::-1], b_repr[:, ::-1]
    emb = jnp.broadcast_to(params['emb'][None, None], x.shape[:2] + (N, EMB))
    feat = jnp.concatenate([fwd_repr, bwd_repr, mask, emb], axis=-1)
    op = params['out']

    fdim = feat.shape[-1]
    fpad = 384
    rows = B * S * N
    feat_flat = feat.reshape(rows, fdim)
    feat_pad = jnp.pad(feat_flat, ((0, 0), (0, fpad - fdim)))
    w1_pad = jnp.pad(op['W1'], ((0, fpad - fdim), (0, 0)))
    w2_pad = jnp.pad(op['W2'], ((0, 0), (0, 128 - C)))
    b2_pad = jnp.pad(op['b2'], ((0, 128 - C),))
    out = _readout(feat_pad, w1_pad, op['b1'], w2_pad, b2_pad)
    imputation = out[:, :C].reshape(B, S, N, C)
    return imputation, (fwd_out, bwd_out, fwd_pred, bwd_pred)


# speedup vs baseline: 3.7750x; 3.7750x over previous
import jax
import jax.numpy as jnp
from jax import lax
from jax.experimental import pallas as pl
from jax.experimental.pallas import tpu as pltpu
from jax.experimental.pallas import tpu_sc as plsc

N = 10000
E = 160000
S = 8
C = 1
H = 64
FF = 128
EMB = 16

NP = 10240
TILES = 16
RPT = NP // TILES
CH = 80
LANES = 128
EPT = CH * LANES
EP = TILES * EPT
NB = 4
RB = NP // NB
FD = 80
FC = 16

_SDS = jax.ShapeDtypeStruct


def _make_hop(M, F, shared):
    mesh = plsc.VectorSubcoreMesh(core_axis_name="c", subcore_axis_name="s")
    grp = F // 16

    def body(x_h, gi_h, si_h, w_h, rc_h, z_h, out_h,
             acc, gi_v, si_v, w_v, gbuf, rbuf, sem):
        c = lax.axis_index("c")
        s = lax.axis_index("s")
        row0 = s * RPT
        rows = pl.ds(row0, RPT)
        pltpu.sync_copy(gi_h.at[c, s], gi_v)
        pltpu.sync_copy(si_h.at[c, s], si_v)
        pltpu.sync_copy(w_h.at[c, s], w_v)
        pltpu.sync_copy(rc_h.at[c, rows], rbuf)
        for m in range(M):
            src = x_h.at[m] if shared else x_h.at[m, c]
            pltpu.sync_copy(z_h.at[rows], acc.at[rows])
            plsc.subcore_barrier()

            @pl.loop(0, CH)
            def _chunks(ch):
                pltpu.async_copy(src.at[gi_v.at[ch]], gbuf, sem).wait()

                @pl.loop(0, LANES // 16)
                def _edges(g):
                    wvec = w_v[ch, pl.ds(g * 16, 16)]
                    for ee in range(16):
                        wv = wvec[ee]
                        e = g * 16 + ee
                        for j in range(grp):
                            sl = pl.ds(j * 16, 16)
                            gbuf[e, sl] = gbuf[e, sl] * wv

                pltpu.sync_copy(gbuf, acc.at[si_v.at[ch]], add=True)

            plsc.subcore_barrier()

            @pl.loop(0, RPT // LANES)
            def _wblk(wb):
                wrows = pl.ds(row0 + wb * LANES, LANES)
                pltpu.sync_copy(acc.at[wrows], gbuf)

                @pl.loop(0, LANES // 16)
                def _rows(g):
                    rvec = rbuf[pl.ds(wb * LANES + g * 16, 16)]
                    for rr in range(16):
                        rc = rvec[rr]
                        r = g * 16 + rr
                        for j in range(grp):
                            sl = pl.ds(j * 16, 16)
                            gbuf[r, sl] = gbuf[r, sl] * rc

                pltpu.sync_copy(gbuf, out_h.at[m, c, wrows])

            if m + 1 < M:
                plsc.subcore_barrier()

    xshape = (M, NP, F) if shared else (M, 2, NP, F)
    return pl.kernel(
        body,
        out_type=_SDS((M, 2, NP, F), jnp.float32),
        mesh=mesh,
        compiler_params=pltpu.CompilerParams(use_tc_tiling_on_sc=False),
        scratch_types=[
            pltpu.VMEM_SHARED((NP, F), jnp.float32),
            pltpu.VMEM((CH, LANES), jnp.int32),
            pltpu.VMEM((CH, LANES), jnp.int32),
            pltpu.VMEM((CH, LANES), jnp.float32),
            pltpu.VMEM((LANES, F), jnp.float32),
            pltpu.VMEM((RPT,), jnp.float32),
            pltpu.SemaphoreType.DMA,
        ],
    ), xshape


_HOP_D1 = _make_hop(1, H, True)[0]
_HOP_80S = _make_hop(2, FD, True)[0]
_HOP_80D = _make_hop(2, FD, False)[0]
_HOP_16S = _make_hop(2, FC, True)[0]
_HOP_16D = _make_hop(2, FC, False)[0]
_HOP_64S = _make_hop(2, H, True)[0]
_HOP_64D = _make_hop(2, H, False)[0]


def _recip_body(deg_ref, rc_ref, sfb_ref):
    for c in range(2):
        d = deg_ref[c, :, 0]
        dm = jnp.maximum(d, 1e-8)
        rc_ref[c, :] = 1.0 / dm
        sfb_ref[c, :] = d / dm


def _tc_recip(deg):
    return pl.pallas_call(
        _recip_body,
        out_shape=[_SDS((2, NP), jnp.float32), _SDS((2, NP), jnp.float32)],
    )(deg)


def _stepA_body(h_ref, x_ref, m_ref, wfs_ref, bfs_ref, xs1_ref, hd_ref):
    h = h_ref[0]
    xs1 = h @ wfs_ref[0] + bfs_ref[0, 0]
    mb = m_ref[0] > 0.5
    x1 = jnp.where(mb, x_ref[0], xs1)
    xs1_ref[0] = xs1
    hd_ref[0] = jnp.concatenate(
        [h, x1, m_ref[0], jnp.zeros((RB, FD - H - 2), jnp.float32)], axis=-1)


def _tc_stepA(hst, xt, mt, wfs, bfs):
    return pl.pallas_call(
        _stepA_body,
        grid=(2, NB),
        in_specs=[
            pl.BlockSpec((1, RB, H), lambda m, i: (m, i, 0)),
            pl.BlockSpec((1, RB, 1), lambda m, i: (m, i, 0)),
            pl.BlockSpec((1, RB, 1), lambda m, i: (m, i, 0)),
            pl.BlockSpec((1, H, 1), lambda m, i: (m, 0, 0)),
            pl.BlockSpec((1, 1, 1), lambda m, i: (m, 0, 0)),
        ],
        out_specs=[pl.BlockSpec((1, RB, 1), lambda m, i: (m, i, 0)),
                   pl.BlockSpec((1, RB, FD), lambda m, i: (m, i, 0))],
        out_shape=[_SDS((2, NP, 1), jnp.float32), _SDS((2, NP, FD), jnp.float32)],
    )(hst, xt, mt, wfs, bfs)


def _stepB_body(pf_ref, pb_ref, h_ref, x_ref, m_ref, sfb_ref,
                wb_ref, vf_ref, vb_ref, bo_ref, pa_ref,
                wro_ref, bro_ref,
                xs2_ref, rep_ref, ci_ref):
    h = h_ref[0]
    feats = jnp.concatenate([pf_ref[0, 0], pb_ref[0, 0], h], axis=-1)
    o = (feats @ wb_ref[0] + bo_ref[0, 0]
         + sfb_ref[0, :][:, None] * vf_ref[0, 0]
         + sfb_ref[1, :][:, None] * vb_ref[0, 0])
    a = pa_ref[0, 0, 0]
    o = jnp.where(o >= 0, o, a * o)
    rep = jnp.concatenate([o, h], axis=-1)
    xs2 = rep @ wro_ref[0] + bro_ref[0, 0]
    mb = m_ref[0] > 0.5
    x2 = jnp.where(mb, x_ref[0], xs2)
    xs2_ref[0] = xs2
    rep_ref[0] = rep
    ci_ref[0] = jnp.concatenate(
        [x2, m_ref[0], jnp.zeros((RB, FC - 2), jnp.float32)], axis=-1)


def _tc_stepB(p1, hst, xt, mt, sfb, wB, vf, vb, bo, pa, wro, bro):
    sfb_blk = pl.BlockSpec((2, RB), lambda m, i: (0, i))
    return pl.pallas_call(
        _stepB_body,
        grid=(2, NB),
        in_specs=[
            pl.BlockSpec((1, 1, RB, FD), lambda m, i: (m, 0, i, 0)),
            pl.BlockSpec((1, 1, RB, FD), lambda m, i: (m, 1, i, 0)),
            pl.BlockSpec((1, RB, H), lambda m, i: (m, i, 0)),
            pl.BlockSpec((1, RB, 1), lambda m, i: (m, i, 0)),
            pl.BlockSpec((1, RB, 1), lambda m, i: (m, i, 0)),
            sfb_blk,
            pl.BlockSpec((1, 2 * FD + H, H), lambda m, i: (m, 0, 0)),
            pl.BlockSpec((1, 1, H), lambda m, i: (m, 0, 0)),
            pl.BlockSpec((1, 1, H), lambda m, i: (m, 0, 0)),
            pl.BlockSpec((1, 1, H), lambda m, i: (m, 0, 0)),
            pl.BlockSpec((1, 1, 1), lambda m, i: (m, 0, 0)),
            pl.BlockSpec((1, 2 * H, 1), lambda m, i: (m, 0, 0)),
            pl.BlockSpec((1, 1, 1), lambda m, i: (m, 0, 0)),
        ],
        out_specs=[pl.BlockSpec((1, RB, 1), lambda m, i: (m, i, 0)),
                   pl.BlockSpec((1, RB, 2 * H), lambda m, i: (m, i, 0)),
                   pl.BlockSpec((1, RB, FC), lambda m, i: (m, i, 0))],
        out_shape=[_SDS((2, NP, 1), jnp.float32),
                   _SDS((2, NP, 2 * H), jnp.float32),
                   _SDS((2, NP, FC), jnp.float32)],
    )(p1, p1, hst, xt, mt, sfb, wB, vf, vb, bo, pa, wro, bro)


def _stepC_body(h_ref, hd1f_ref, hd1b_ref, hd2f_ref, hd2b_ref,
                ci_ref, c1f_ref, c1b_ref, c2f_ref, c2b_ref,
                w_ref, b_ref, ru_ref, rh_ref):
    h = h_ref[0]
    feats = jnp.concatenate(
        [h, hd1f_ref[0, 0, :, :H], hd2f_ref[0, 0, :, :H],
         hd1b_ref[0, 0, :, :H], hd2b_ref[0, 0, :, :H],
         ci_ref[0], c1f_ref[0, 0], c2f_ref[0, 0],
         c1b_ref[0, 0], c2b_ref[0, 0]], axis=-1)
    pre = feats @ w_ref[0] + b_ref[0, 0]
    ru = jax.nn.sigmoid(pre)
    ru_ref[0] = ru
    rh_ref[0] = ru[:, :H] * h


def _tc_stepC(hst, p1a, p1b, ci, p2a, p2b, wC, bC):
    kdim = 5 * H + 5 * FC
    bfd = lambda cix: pl.BlockSpec((1, 1, RB, FD), lambda m, i, c=cix: (m, c, i, 0))
    bfc = lambda cix: pl.BlockSpec((1, 1, RB, FC), lambda m, i, c=cix: (m, c, i, 0))
    return pl.pallas_call(
        _stepC_body,
        grid=(2, NB),
        in_specs=[
            pl.BlockSpec((1, RB, H), lambda m, i: (m, i, 0)),
            bfd(0), bfd(1), bfd(0), bfd(1),
            pl.BlockSpec((1, RB, FC), lambda m, i: (m, i, 0)),
            bfc(0), bfc(1), bfc(0), bfc(1),
            pl.BlockSpec((1, kdim, 2 * H), lambda m, i: (m, 0, 0)),
            pl.BlockSpec((1, 1, 2 * H), lambda m, i: (m, 0, 0)),
        ],
        out_specs=[pl.BlockSpec((1, RB, 2 * H), lambda m, i: (m, i, 0)),
                   pl.BlockSpec((1, RB, H), lambda m, i: (m, i, 0))],
        out_shape=[_SDS((2, NP, 2 * H), jnp.float32),
                   _SDS((2, NP, H), jnp.float32)],
    )(hst, p1a, p1a, p1b, p1b, ci, p2a, p2a, p2b, p2b, wC, bC)


def _stepD_body(h_ref, ru_ref, rh_ref, r1f_ref, r1b_ref, r2f_ref, r2b_ref,
                ci_ref, c1f_ref, c1b_ref, c2f_ref, c2b_ref,
                w_ref, b_ref, hn_ref):
    h = h_ref[0]
    feats = jnp.concatenate(
        [rh_ref[0], r1f_ref[0, 0], r2f_ref[0, 0], r1b_ref[0, 0], r2b_ref[0, 0],
         ci_ref[0], c1f_ref[0, 0], c2f_ref[0, 0],
         c1b_ref[0, 0], c2b_ref[0, 0]], axis=-1)
    cc = jnp.tanh(feats @ w_ref[0] + b_ref[0, 0])
    u = ru_ref[0, :, H:]
    hn_ref[0] = u * h + (1.0 - u) * cc


def _tc_stepD(hst, ru, rh, p3a, p3b, ci, p2a, p2b, wD, bD):
    kdim = 5 * H + 5 * FC
    bh = lambda cix: pl.BlockSpec((1, 1, RB, H), lambda m, i, c=cix: (m, c, i, 0))
    bfc = lambda cix: pl.BlockSpec((1, 1, RB, FC), lambda m, i, c=cix: (m, c, i, 0))
    return pl.pallas_call(
        _stepD_body,
        grid=(2, NB),
        in_specs=[
            pl.BlockSpec((1, RB, H), lambda m, i: (m, i, 0)),
            pl.BlockSpec((1, RB, 2 * H), lambda m, i: (m, i, 0)),
            pl.BlockSpec((1, RB, H), lambda m, i: (m, i, 0)),
            bh(0), bh(1), bh(0), bh(1),
            pl.BlockSpec((1, RB, FC), lambda m, i: (m, i, 0)),
            bfc(0), bfc(1), bfc(0), bfc(1),
            pl.BlockSpec((1, kdim, H), lambda m, i: (m, 0, 0)),
            pl.BlockSpec((1, 1, H), lambda m, i: (m, 0, 0)),
        ],
        out_specs=pl.BlockSpec((1, RB, H), lambda m, i: (m, i, 0)),
        out_shape=_SDS((2, NP, H), jnp.float32),
    )(hst, ru, rh, p3a, p3a, p3b, p3b, ci, p2a, p2a, p2b, p2b, wD, bD)


def _read_body(rf_ref, rb_ref, m_ref, e_ref, w1_ref, b1_ref, w2_ref, b2_ref,
               out_ref):
    feats = jnp.concatenate(
        [rf_ref[0, 0], rb_ref[0, 0], m_ref[0], e_ref[...]], axis=-1)
    hid = jnp.maximum(feats @ w1_ref[...] + b1_ref[...], 0.0)
    out_ref[0] = hid @ w2_ref[...] + b2_ref[...]


def _tc_read(rep, mpad, emb, w1, b1, w2, b2):
    kdim = 4 * H + 1 + EMB
    return pl.pallas_call(
        _read_body,
        grid=(S, NB),
        in_specs=[
            pl.BlockSpec((1, 1, RB, 2 * H), lambda t, i: (t, 0, i, 0)),
            pl.BlockSpec((1, 1, RB, 2 * H), lambda t, i: (S - 1 - t, 1, i, 0)),
            pl.BlockSpec((1, RB, 1), lambda t, i: (t, i, 0)),
            pl.BlockSpec((RB, EMB), lambda t, i: (i, 0)),
            pl.BlockSpec((kdim, FF), lambda t, i: (0, 0)),
            pl.BlockSpec((FF,), lambda t, i: (0,)),
            pl.BlockSpec((FF, 128), lambda t, i: (0, 0)),
            pl.BlockSpec((128,), lambda t, i: (0,)),
        ],
        out_specs=pl.BlockSpec((1, RB, 128), lambda t, i: (t, i, 0)),
        out_shape=_SDS((S, NP, 128), jnp.float32),
    )(rep, rep, mpad, emb, w1, b1, w2, b2)


def _prep_model(gp):
    dp = gp['dec']
    wli, bli = dp['lin_in']['W'], dp['lin_in']['b']
    wgf, wgb, bgc = dp['gc']['Wf'], dp['gc']['Wb'], dp['gc']['b']
    wlo, blo = dp['lin_out']['W'], dp['lin_out']['b']
    wz = jnp.concatenate([wli[2:], wli[:2], jnp.zeros((FD - H - 2, H))], 0)
    a_f = wz @ wgf @ wlo[:H]
    a_b = wz @ wgb @ wlo[:H]
    wB = jnp.concatenate([a_f, a_b, wlo[H:]], axis=0)
    vf = bli @ wgf @ wlo[:H]
    vb = bli @ wgb @ wlo[:H]
    bo = bgc @ wlo[:H] + blo

    def conv_w(p, fpart):
        def xpad(w2):
            return jnp.concatenate([w2, jnp.zeros((FC - 2, w2.shape[1]))], 0)
        return jnp.concatenate([
            fpart(p['W0']), fpart(p['Wf'][0]), fpart(p['Wf'][1]),
            fpart(p['Wb'][0]), fpart(p['Wb'][1]),
            xpad(p['W0'][:2]), xpad(p['Wf'][0][:2]), xpad(p['Wf'][1][:2]),
            xpad(p['Wb'][0][:2]), xpad(p['Wb'][1][:2])], axis=0)

    cr, cu, cc = gp['cell']['r'], gp['cell']['u'], gp['cell']['c']
    wC = jnp.concatenate([conv_w(cr, lambda w: w[2:]),
                          conv_w(cu, lambda w: w[2:])], axis=1)
    bC = jnp.concatenate([cr['b'], cu['b']])
    wD = conv_w(cc, lambda w: w[2:])
    bD = cc['b']
    return {
        'wfs': gp['first_stage']['W'],
        'wB': wB, 'vf': vf, 'vb': vb, 'bo': bo,
        'wro': dp['read_out']['W'],
        'wC': wC, 'bC': bC, 'wD': wD, 'bD': bD,
        'h0': gp['h0'],
    }


def _pad_rows(a, np_=NP):
    return jnp.pad(a, ((0, np_ - a.shape[0]),) + ((0, 0),) * (a.ndim - 1))


def kernel(x, edge_index, edge_weight, mask, params):
    f32 = jnp.float32
    src, dst = edge_index[0], edge_index[1]
    padE = EP - E

    def pack(g, s_, w_):
        g = jnp.pad(g, (0, padE)).reshape(TILES, CH, LANES)
        s_ = jnp.pad(s_, (0, padE)).reshape(TILES, CH, LANES)
        w_ = jnp.pad(w_, (0, padE)).reshape(TILES, CH, LANES)
        return g, s_, w_

    g0, s0, w0 = pack(dst, src, edge_weight)
    g1, s1, w1 = pack(src, dst, edge_weight)
    GI = jnp.stack([g0, g1])
    SI = jnp.stack([s0, s1])
    WE = jnp.stack([w0, w1]).astype(f32)

    zeros80 = jnp.zeros((NP, FD), f32)
    zeros64 = jnp.zeros((NP, H), f32)
    zeros16 = jnp.zeros((NP, FC), f32)
    ones_rc = jnp.ones((2, NP), f32)

    deg = _HOP_D1(jnp.ones((1, NP, H), f32), GI, SI, WE, ones_rc, zeros64)
    recip, sfb = _tc_recip(deg[0])

    pf = _prep_model(params['fwd'])
    pb = _prep_model(params['bwd'])

    def st(k):
        return jnp.stack([pf[k], pb[k]])

    Wfs = st('wfs')
    Bfs = jnp.stack([params['fwd']['first_stage']['b'],
                     params['bwd']['first_stage']['b']])[:, None]
    WB = st('wB')
    Vf, Vb, Bo = st('vf')[:, None], st('vb')[:, None], st('bo')[:, None]
    Pa = jnp.stack([params['fwd']['dec']['prelu_a'],
                    params['bwd']['dec']['prelu_a']])[:, None, None]
    Wro = st('wro')
    Bro = jnp.stack([params['fwd']['dec']['read_out']['b'],
                     params['bwd']['dec']['read_out']['b']])[:, None]
    WC, WD = st('wC'), st('wD')
    BC, BD = st('bC')[:, None], st('bD')[:, None]

    x_pad = jnp.pad(x[0], ((0, 0), (0, NP - N), (0, 0)))
    m_pad = jnp.pad(mask[0], ((0, 0), (0, NP - N), (0, 0)))
    xs_st = jnp.stack([x_pad, x_pad[::-1]], axis=1)
    ms_st = jnp.stack([m_pad, m_pad[::-1]], axis=1)

    hst = jnp.stack([_pad_rows(pf['h0']), _pad_rows(pb['h0'])])

    xs1_l, xs2_l, rep_l = [], [], []
    for t in range(S):
        xt, mt = xs_st[t], ms_st[t]
        xs1, hd = _tc_stepA(hst, xt, mt, Wfs, Bfs)
        p1a = _HOP_80S(hd, GI, SI, WE, recip, zeros80)
        p1b = _HOP_80D(p1a, GI, SI, WE, recip, zeros80)
        xs2, rep, ci = _tc_stepB(p1a, hst, xt, mt, sfb,
                                 WB, Vf, Vb, Bo, Pa, Wro, Bro)
        p2a = _HOP_16S(ci, GI, SI, WE, recip, zeros16)
        p2b = _HOP_16D(p2a, GI, SI, WE, recip, zeros16)
        ru, rh = _tc_stepC(hst, p1a, p1b, ci, p2a, p2b, WC, BC)
        p3a = _HOP_64S(rh, GI, SI, WE, recip, zeros64)
        p3b = _HOP_64D(p3a, GI, SI, WE, recip, zeros64)
        hst = _tc_stepD(hst, ru, rh, p3a, p3b, ci, p2a, p2b, WD, BD)
        xs1_l.append(xs1)
        xs2_l.append(xs2)
        rep_l.append(rep)

    xs1_s = jnp.stack(xs1_l)
    xs2_s = jnp.stack(xs2_l)
    rep_s = jnp.stack(rep_l)

    emb_pad = _pad_rows(params['emb'])
    op = params['out']
    w2_pad = jnp.pad(op['W2'], ((0, 0), (0, 128 - C)))
    b2_pad = jnp.pad(op['b2'], ((0, 128 - C),))
    outr = _tc_read(rep_s, m_pad, emb_pad, op['W1'], op['b1'], w2_pad, b2_pad)

    imputation = outr[:, :N, :C][None]
    fwd_out = xs2_s[:, 0, :N][None]
    bwd_out = xs2_s[::-1, 1, :N][None]
    fwd_pred = xs1_s[:, 0, :N][None]
    bwd_pred = xs1_s[::-1, 1, :N][None]
    return imputation, (fwd_out, bwd_out, fwd_pred, bwd_pred)


# speedup vs baseline: 5.7334x; 1.5188x over previous
import jax
import jax.numpy as jnp
from jax import lax
from jax.experimental import pallas as pl
from jax.experimental.pallas import tpu as pltpu
from jax.experimental.pallas import tpu_sc as plsc

N = 10000
E = 160000
S = 8
C = 1
H = 64
FF = 128
EMB = 16

NP = 10240
TILES = 16
RPT = NP // TILES
LANES = 128
CH = 80
NSLOT = 4
QUADS = CH // NSLOT
EPT = CH * LANES
EP = TILES * EPT
NB = 4
RB = NP // NB
FD = 80
FC = 16

_SDS = jax.ShapeDtypeStruct


def _make_hop(M, F, shared):
    mesh = plsc.VectorSubcoreMesh(core_axis_name="c", subcore_axis_name="s")
    grp = F // 16

    def body(x_h, gi_h, si_h, w_h, rc_h, z_h, out_h,
             acc, gi_v, si_v, w_v, gbuf, rbuf, sem0, sem1, sem2, sem3):
        c = lax.axis_index("c")
        s = lax.axis_index("s")
        row0 = s * RPT
        rows = pl.ds(row0, RPT)
        pltpu.sync_copy(gi_h.at[c, s], gi_v)
        pltpu.sync_copy(si_h.at[c, s], si_v)
        pltpu.sync_copy(w_h.at[c, s], w_v)
        pltpu.sync_copy(rc_h.at[c, rows], rbuf)
        sems = (sem0, sem1, sem2, sem3)

        def scale(b, ch):
            @pl.loop(0, LANES // 16)
            def _edges(g):
                wvec = w_v[ch, pl.ds(g * 16, 16)]
                for ee in range(16):
                    wv = wvec[ee]
                    e = g * 16 + ee
                    for j in range(grp):
                        sl = pl.ds(j * 16, 16)
                        gbuf[b, e, sl] = gbuf[b, e, sl] * wv

        for m in range(M):
            src = x_h.at[m] if shared else x_h.at[m, c]

            def issue(ch, b):
                pltpu.async_copy(src.at[gi_v.at[ch]], gbuf.at[b], sems[b])

            def drain(ch, b):
                pltpu.make_async_copy(
                    src.at[gi_v.at[ch]], gbuf.at[b], sems[b]).wait()

            for k in range(NSLOT - 1):
                issue(k, k)
            pltpu.sync_copy(z_h.at[rows], acc.at[rows])
            plsc.subcore_barrier()

            @pl.loop(0, QUADS)
            def _quads(j):
                ch0 = j * NSLOT
                for k in range(NSLOT):
                    ch = ch0 + k

                    @pl.when(ch + NSLOT - 1 < CH)
                    def _pre():
                        issue(ch + NSLOT - 1, (k + NSLOT - 1) % NSLOT)

                    drain(ch, k)
                    scale(k, ch)
                    pltpu.sync_copy(gbuf.at[k], acc.at[si_v.at[ch]], add=True)

            plsc.subcore_barrier()

            @pl.loop(0, RPT // LANES)
            def _wblk(wb):
                wrows = pl.ds(row0 + wb * LANES, LANES)
                pltpu.sync_copy(acc.at[wrows], gbuf.at[0])

                @pl.loop(0, LANES // 16)
                def _rows(g):
                    rvec = rbuf[pl.ds(wb * LANES + g * 16, 16)]
                    for rr in range(16):
                        rc = rvec[rr]
                        r = g * 16 + rr
                        for j in range(grp):
                            sl = pl.ds(j * 16, 16)
                            gbuf[0, r, sl] = gbuf[0, r, sl] * rc

                pltpu.sync_copy(gbuf.at[0], out_h.at[m, c, wrows])

            if m + 1 < M:
                plsc.subcore_barrier()

    xshape = (M, NP, F) if shared else (M, 2, NP, F)
    return pl.kernel(
        body,
        out_type=_SDS((M, 2, NP, F), jnp.float32),
        mesh=mesh,
        compiler_params=pltpu.CompilerParams(use_tc_tiling_on_sc=False),
        scratch_types=[
            pltpu.VMEM_SHARED((NP, F), jnp.float32),
            pltpu.VMEM((CH, LANES), jnp.int32),
            pltpu.VMEM((CH, LANES), jnp.int32),
            pltpu.VMEM((CH, LANES), jnp.float32),
            pltpu.VMEM((NSLOT, LANES, F), jnp.float32),
            pltpu.VMEM((RPT,), jnp.float32),
            pltpu.SemaphoreType.DMA,
            pltpu.SemaphoreType.DMA,
            pltpu.SemaphoreType.DMA,
            pltpu.SemaphoreType.DMA,
        ],
    ), xshape


_HOP_D1 = _make_hop(1, FC, True)[0]
_HOP_80S = _make_hop(2, FD, True)[0]
_HOP_80D = _make_hop(2, FD, False)[0]
_HOP_16S = _make_hop(2, FC, True)[0]
_HOP_16D = _make_hop(2, FC, False)[0]
_HOP_64S = _make_hop(2, H, True)[0]
_HOP_64D = _make_hop(2, H, False)[0]


def _recip_body(deg_ref, rc_ref, sfb_ref):
    for c in range(2):
        d = deg_ref[c, :, 0]
        dm = jnp.maximum(d, 1e-8)
        rc_ref[c, :] = 1.0 / dm
        sfb_ref[c, :] = d / dm


def _tc_recip(deg):
    return pl.pallas_call(
        _recip_body,
        out_shape=[_SDS((2, NP), jnp.float32), _SDS((2, NP), jnp.float32)],
    )(deg)


def _stepA_body(h_ref, x_ref, m_ref, wfs_ref, bfs_ref, xs1_ref, hd_ref):
    h = h_ref[0]
    xs1 = h @ wfs_ref[0] + bfs_ref[0, 0]
    mb = m_ref[0] > 0.5
    x1 = jnp.where(mb, x_ref[0], xs1)
    xs1_ref[0] = xs1
    hd_ref[0] = jnp.concatenate(
        [h, x1, m_ref[0], jnp.zeros((RB, FD - H - 2), jnp.float32)], axis=-1)


def _tc_stepA(hst, xt, mt, wfs, bfs):
    return pl.pallas_call(
        _stepA_body,
        grid=(2, NB),
        in_specs=[
            pl.BlockSpec((1, RB, H), lambda m, i: (m, i, 0)),
            pl.BlockSpec((1, RB, 1), lambda m, i: (m, i, 0)),
            pl.BlockSpec((1, RB, 1), lambda m, i: (m, i, 0)),
            pl.BlockSpec((1, H, 1), lambda m, i: (m, 0, 0)),
            pl.BlockSpec((1, 1, 1), lambda m, i: (m, 0, 0)),
        ],
        out_specs=[pl.BlockSpec((1, RB, 1), lambda m, i: (m, i, 0)),
                   pl.BlockSpec((1, RB, FD), lambda m, i: (m, i, 0))],
        out_shape=[_SDS((2, NP, 1), jnp.float32), _SDS((2, NP, FD), jnp.float32)],
    )(hst, xt, mt, wfs, bfs)


def _stepB_body(pf_ref, pb_ref, h_ref, x_ref, m_ref, sfb_ref,
                wb_ref, vf_ref, vb_ref, bo_ref, pa_ref,
                wro_ref, bro_ref,
                xs2_ref, rep_ref, ci_ref):
    h = h_ref[0]
    feats = jnp.concatenate([pf_ref[0, 0], pb_ref[0, 0], h], axis=-1)
    o = (feats @ wb_ref[0] + bo_ref[0, 0]
         + sfb_ref[0, :][:, None] * vf_ref[0, 0]
         + sfb_ref[1, :][:, None] * vb_ref[0, 0])
    a = pa_ref[0, 0, 0]
    o = jnp.where(o >= 0, o, a * o)
    rep = jnp.concatenate([o, h], axis=-1)
    xs2 = rep @ wro_ref[0] + bro_ref[0, 0]
    mb = m_ref[0] > 0.5
    x2 = jnp.where(mb, x_ref[0], xs2)
    xs2_ref[0] = xs2
    rep_ref[0] = rep
    ci_ref[0] = jnp.concatenate(
        [x2, m_ref[0], jnp.zeros((RB, FC - 2), jnp.float32)], axis=-1)


def _tc_stepB(p1, hst, xt, mt, sfb, wB, vf, vb, bo, pa, wro, bro):
    sfb_blk = pl.BlockSpec((2, RB), lambda m, i: (0, i))
    return pl.pallas_call(
        _stepB_body,
        grid=(2, NB),
        in_specs=[
            pl.BlockSpec((1, 1, RB, FD), lambda m, i: (m, 0, i, 0)),
            pl.BlockSpec((1, 1, RB, FD), lambda m, i: (m, 1, i, 0)),
            pl.BlockSpec((1, RB, H), lambda m, i: (m, i, 0)),
            pl.BlockSpec((1, RB, 1), lambda m, i: (m, i, 0)),
            pl.BlockSpec((1, RB, 1), lambda m, i: (m, i, 0)),
            sfb_blk,
            pl.BlockSpec((1, 2 * FD + H, H), lambda m, i: (m, 0, 0)),
            pl.BlockSpec((1, 1, H), lambda m, i: (m, 0, 0)),
            pl.BlockSpec((1, 1, H), lambda m, i: (m, 0, 0)),
            pl.BlockSpec((1, 1, H), lambda m, i: (m, 0, 0)),
            pl.BlockSpec((1, 1, 1), lambda m, i: (m, 0, 0)),
            pl.BlockSpec((1, 2 * H, 1), lambda m, i: (m, 0, 0)),
            pl.BlockSpec((1, 1, 1), lambda m, i: (m, 0, 0)),
        ],
        out_specs=[pl.BlockSpec((1, RB, 1), lambda m, i: (m, i, 0)),
                   pl.BlockSpec((1, RB, 2 * H), lambda m, i: (m, i, 0)),
                   pl.BlockSpec((1, RB, FC), lambda m, i: (m, i, 0))],
        out_shape=[_SDS((2, NP, 1), jnp.float32),
                   _SDS((2, NP, 2 * H), jnp.float32),
                   _SDS((2, NP, FC), jnp.float32)],
    )(p1, p1, hst, xt, mt, sfb, wB, vf, vb, bo, pa, wro, bro)


def _stepC_body(h_ref, hd1f_ref, hd1b_ref, hd2f_ref, hd2b_ref,
                ci_ref, c1f_ref, c1b_ref, c2f_ref, c2b_ref,
                w_ref, b_ref, ru_ref, rh_ref):
    h = h_ref[0]
    feats = jnp.concatenate(
        [h, hd1f_ref[0, 0, :, :H], hd2f_ref[0, 0, :, :H],
         hd1b_ref[0, 0, :, :H], hd2b_ref[0, 0, :, :H],
         ci_ref[0], c1f_ref[0, 0], c2f_ref[0, 0],
         c1b_ref[0, 0], c2b_ref[0, 0]], axis=-1)
    pre = feats @ w_ref[0] + b_ref[0, 0]
    ru = jax.nn.sigmoid(pre)
    ru_ref[0] = ru
    rh_ref[0] = ru[:, :H] * h


def _tc_stepC(hst, p1a, p1b, ci, p2a, p2b, wC, bC):
    kdim = 5 * H + 5 * FC
    bfd = lambda cix: pl.BlockSpec((1, 1, RB, FD), lambda m, i, c=cix: (m, c, i, 0))
    bfc = lambda cix: pl.BlockSpec((1, 1, RB, FC), lambda m, i, c=cix: (m, c, i, 0))
    return pl.pallas_call(
        _stepC_body,
        grid=(2, NB),
        in_specs=[
            pl.BlockSpec((1, RB, H), lambda m, i: (m, i, 0)),
            bfd(0), bfd(1), bfd(0), bfd(1),
            pl.BlockSpec((1, RB, FC), lambda m, i: (m, i, 0)),
            bfc(0), bfc(1), bfc(0), bfc(1),
            pl.BlockSpec((1, kdim, 2 * H), lambda m, i: (m, 0, 0)),
            pl.BlockSpec((1, 1, 2 * H), lambda m, i: (m, 0, 0)),
        ],
        out_specs=[pl.BlockSpec((1, RB, 2 * H), lambda m, i: (m, i, 0)),
                   pl.BlockSpec((1, RB, H), lambda m, i: (m, i, 0))],
        out_shape=[_SDS((2, NP, 2 * H), jnp.float32),
                   _SDS((2, NP, H), jnp.float32)],
    )(hst, p1a, p1a, p1b, p1b, ci, p2a, p2a, p2b, p2b, wC, bC)


def _stepD_body(h_ref, ru_ref, rh_ref, r1f_ref, r1b_ref, r2f_ref, r2b_ref,
                ci_ref, c1f_ref, c1b_ref, c2f_ref, c2b_ref,
                w_ref, b_ref, hn_ref):
    h = h_ref[0]
    feats = jnp.concatenate(
        [rh_ref[0], r1f_ref[0, 0], r2f_ref[0, 0], r1b_ref[0, 0], r2b_ref[0, 0],
         ci_ref[0], c1f_ref[0, 0], c2f_ref[0, 0],
         c1b_ref[0, 0], c2b_ref[0, 0]], axis=-1)
    cc = jnp.tanh(feats @ w_ref[0] + b_ref[0, 0])
    u = ru_ref[0, :, H:]
    hn_ref[0] = u * h + (1.0 - u) * cc


def _tc_stepD(hst, ru, rh, p3a, p3b, ci, p2a, p2b, wD, bD):
    kdim = 5 * H + 5 * FC
    bh = lambda cix: pl.BlockSpec((1, 1, RB, H), lambda m, i, c=cix: (m, c, i, 0))
    bfc = lambda cix: pl.BlockSpec((1, 1, RB, FC), lambda m, i, c=cix: (m, c, i, 0))
    return pl.pallas_call(
        _stepD_body,
        grid=(2, NB),
        in_specs=[
            pl.BlockSpec((1, RB, H), lambda m, i: (m, i, 0)),
            pl.BlockSpec((1, RB, 2 * H), lambda m, i: (m, i, 0)),
            pl.BlockSpec((1, RB, H), lambda m, i: (m, i, 0)),
            bh(0), bh(1), bh(0), bh(1),
            pl.BlockSpec((1, RB, FC), lambda m, i: (m, i, 0)),
            bfc(0), bfc(1), bfc(0), bfc(1),
            pl.BlockSpec((1, kdim, H), lambda m, i: (m, 0, 0)),
            pl.BlockSpec((1, 1, H), lambda m, i: (m, 0, 0)),
        ],
        out_specs=pl.BlockSpec((1, RB, H), lambda m, i: (m, i, 0)),
        out_shape=_SDS((2, NP, H), jnp.float32),
    )(hst, ru, rh, p3a, p3a, p3b, p3b, ci, p2a, p2a, p2b, p2b, wD, bD)


def _read_body(rf_ref, rb_ref, m_ref, e_ref, w1_ref, b1_ref, w2_ref, b2_ref,
               out_ref):
    feats = jnp.concatenate(
        [rf_ref[0, 0], rb_ref[0, 0], m_ref[0], e_ref[...]], axis=-1)
    hid = jnp.maximum(feats @ w1_ref[...] + b1_ref[...], 0.0)
    out_ref[0] = hid @ w2_ref[...] + b2_ref[...]


def _tc_read(rep, mpad, emb, w1, b1, w2, b2):
    kdim = 4 * H + 1 + EMB
    return pl.pallas_call(
        _read_body,
        grid=(S, NB),
        in_specs=[
            pl.BlockSpec((1, 1, RB, 2 * H), lambda t, i: (t, 0, i, 0)),
            pl.BlockSpec((1, 1, RB, 2 * H), lambda t, i: (S - 1 - t, 1, i, 0)),
            pl.BlockSpec((1, RB, 1), lambda t, i: (t, i, 0)),
            pl.BlockSpec((RB, EMB), lambda t, i: (i, 0)),
            pl.BlockSpec((kdim, FF), lambda t, i: (0, 0)),
            pl.BlockSpec((FF,), lambda t, i: (0,)),
            pl.BlockSpec((FF, 128), lambda t, i: (0, 0)),
            pl.BlockSpec((128,), lambda t, i: (0,)),
        ],
        out_specs=pl.BlockSpec((1, RB, 128), lambda t, i: (t, i, 0)),
        out_shape=_SDS((S, NP, 128), jnp.float32),
    )(rep, rep, mpad, emb, w1, b1, w2, b2)


def _prep_model(gp):
    dp = gp['dec']
    wli, bli = dp['lin_in']['W'], dp['lin_in']['b']
    wgf, wgb, bgc = dp['gc']['Wf'], dp['gc']['Wb'], dp['gc']['b']
    wlo, blo = dp['lin_out']['W'], dp['lin_out']['b']
    wz = jnp.concatenate([wli[2:], wli[:2], jnp.zeros((FD - H - 2, H))], 0)
    a_f = wz @ wgf @ wlo[:H]
    a_b = wz @ wgb @ wlo[:H]
    wB = jnp.concatenate([a_f, a_b, wlo[H:]], axis=0)
    vf = bli @ wgf @ wlo[:H]
    vb = bli @ wgb @ wlo[:H]
    bo = bgc @ wlo[:H] + blo

    def conv_w(p, fpart):
        def xpad(w2):
            return jnp.concatenate([w2, jnp.zeros((FC - 2, w2.shape[1]))], 0)
        return jnp.concatenate([
            fpart(p['W0']), fpart(p['Wf'][0]), fpart(p['Wf'][1]),
            fpart(p['Wb'][0]), fpart(p['Wb'][1]),
            xpad(p['W0'][:2]), xpad(p['Wf'][0][:2]), xpad(p['Wf'][1][:2]),
            xpad(p['Wb'][0][:2]), xpad(p['Wb'][1][:2])], axis=0)

    cr, cu, cc = gp['cell']['r'], gp['cell']['u'], gp['cell']['c']
    wC = jnp.concatenate([conv_w(cr, lambda w: w[2:]),
                          conv_w(cu, lambda w: w[2:])], axis=1)
    bC = jnp.concatenate([cr['b'], cu['b']])
    wD = conv_w(cc, lambda w: w[2:])
    bD = cc['b']
    return {
        'wfs': gp['first_stage']['W'],
        'wB': wB, 'vf': vf, 'vb': vb, 'bo': bo,
        'wro': dp['read_out']['W'],
        'wC': wC, 'bC': bC, 'wD': wD, 'bD': bD,
        'h0': gp['h0'],
    }


def _pad_rows(a, np_=NP):
    return jnp.pad(a, ((0, np_ - a.shape[0]),) + ((0, 0),) * (a.ndim - 1))


def kernel(x, edge_index, edge_weight, mask, params):
    f32 = jnp.float32
    src, dst = edge_index[0], edge_index[1]
    padE = EP - E

    def pack(g, s_, w_):
        g = jnp.pad(g, (0, padE)).reshape(TILES, CH, LANES)
        s_ = jnp.pad(s_, (0, padE)).reshape(TILES, CH, LANES)
        w_ = jnp.pad(w_, (0, padE)).reshape(TILES, CH, LANES)
        return g, s_, w_

    g0, s0, w0 = pack(dst, src, edge_weight)
    g1, s1, w1 = pack(src, dst, edge_weight)
    GI = jnp.stack([g0, g1])
    SI = jnp.stack([s0, s1])
    WE = jnp.stack([w0, w1]).astype(f32)

    zeros80 = jnp.zeros((NP, FD), f32)
    zeros64 = jnp.zeros((NP, H), f32)
    zeros16 = jnp.zeros((NP, FC), f32)
    ones_rc = jnp.ones((2, NP), f32)

    deg = _HOP_D1(jnp.ones((1, NP, FC), f32), GI, SI, WE, ones_rc, zeros16)
    recip, sfb = _tc_recip(deg[0])

    pf = _prep_model(params['fwd'])
    pb = _prep_model(params['bwd'])

    def st(k):
        return jnp.stack([pf[k], pb[k]])

    Wfs = st('wfs')
    Bfs = jnp.stack([params['fwd']['first_stage']['b'],
                     params['bwd']['first_stage']['b']])[:, None]
    WB = st('wB')
    Vf, Vb, Bo = st('vf')[:, None], st('vb')[:, None], st('bo')[:, None]
    Pa = jnp.stack([params['fwd']['dec']['prelu_a'],
                    params['bwd']['dec']['prelu_a']])[:, None, None]
    Wro = st('wro')
    Bro = jnp.stack([params['fwd']['dec']['read_out']['b'],
                     params['bwd']['dec']['read_out']['b']])[:, None]
    WC, WD = st('wC'), st('wD')
    BC, BD = st('bC')[:, None], st('bD')[:, None]

    x_pad = jnp.pad(x[0], ((0, 0), (0, NP - N), (0, 0)))
    m_pad = jnp.pad(mask[0], ((0, 0), (0, NP - N), (0, 0)))
    xs_st = jnp.stack([x_pad, x_pad[::-1]], axis=1)
    ms_st = jnp.stack([m_pad, m_pad[::-1]], axis=1)

    hst = jnp.stack([_pad_rows(pf['h0']), _pad_rows(pb['h0'])])

    xs1_l, xs2_l, rep_l = [], [], []
    for t in range(S):
        xt, mt = xs_st[t], ms_st[t]
        xs1, hd = _tc_stepA(hst, xt, mt, Wfs, Bfs)
        p1a = _HOP_80S(hd, GI, SI, WE, recip, zeros80)
        p1b = _HOP_80D(p1a, GI, SI, WE, recip, zeros80)
        xs2, rep, ci = _tc_stepB(p1a, hst, xt, mt, sfb,
                                 WB, Vf, Vb, Bo, Pa, Wro, Bro)
        p2a = _HOP_16S(ci, GI, SI, WE, recip, zeros16)
        p2b = _HOP_16D(p2a, GI, SI, WE, recip, zeros16)
        ru, rh = _tc_stepC(hst, p1a, p1b, ci, p2a, p2b, WC, BC)
        p3a = _HOP_64S(rh, GI, SI, WE, recip, zeros64)
        p3b = _HOP_64D(p3a, GI, SI, WE, recip, zeros64)
        hst = _tc_stepD(hst, ru, rh, p3a, p3b, ci, p2a, p2b, WD, BD)
        xs1_l.append(xs1)
        xs2_l.append(xs2)
        rep_l.append(rep)

    xs1_s = jnp.stack(xs1_l)
    xs2_s = jnp.stack(xs2_l)
    rep_s = jnp.stack(rep_l)

    emb_pad = _pad_rows(params['emb'])
    op = params['out']
    w2_pad = jnp.pad(op['W2'], ((0, 0), (0, 128 - C)))
    b2_pad = jnp.pad(op['b2'], ((0, 128 - C),))
    outr = _tc_read(rep_s, m_pad, emb_pad, op['W1'], op['b1'], w2_pad, b2_pad)

    imputation = outr[:, :N, :C][None]
    fwd_out = xs2_s[:, 0, :N][None]
    bwd_out = xs2_s[::-1, 1, :N][None]
    fwd_pred = xs1_s[:, 0, :N][None]
    bwd_pred = xs1_s[::-1, 1, :N][None]
    return imputation, (fwd_out, bwd_out, fwd_pred, bwd_pred)


# speedup vs baseline: 5.7337x; 1.0000x over previous
import jax
import jax.numpy as jnp
from jax import lax
from jax.experimental import pallas as pl
from jax.experimental.pallas import tpu as pltpu
from jax.experimental.pallas import tpu_sc as plsc

N = 10000
E = 160000
S = 8
C = 1
H = 64
FF = 128
EMB = 16

NP = 10240
TILES = 16
RPT = NP // TILES
LANES = 128
CH = 80
NSLOT = 4
QUADS = CH // NSLOT
EPT = CH * LANES
EP = TILES * EPT
NB = 4
RB = NP // NB
FD = 80
FC = 16

_SDS = jax.ShapeDtypeStruct


def _make_hop(M, F, shared):
    mesh = plsc.VectorSubcoreMesh(core_axis_name="c", subcore_axis_name="s")
    grp = F // 16

    def body(x_h, gi_h, si_h, w_h, rc_h, z_h, out_h,
             acc, gi_v, si_v, w_v, gbuf, rbuf, sem0, sem1, sem2, sem3,
             ssem0, ssem1, ssem2, ssem3):
        c = lax.axis_index("c")
        s = lax.axis_index("s")
        row0 = s * RPT
        rows = pl.ds(row0, RPT)
        pltpu.sync_copy(gi_h.at[c, s], gi_v)
        pltpu.sync_copy(si_h.at[c, s], si_v)
        pltpu.sync_copy(w_h.at[c, s], w_v)
        pltpu.sync_copy(rc_h.at[c, rows], rbuf)
        sems = (sem0, sem1, sem2, sem3)
        ssems = (ssem0, ssem1, ssem2, ssem3)

        def scale(b, ch):
            @pl.loop(0, LANES // 16)
            def _edges(g):
                wvec = w_v[ch, pl.ds(g * 16, 16)]
                for ee in range(16):
                    wv = wvec[ee]
                    e = g * 16 + ee
                    for j in range(grp):
                        sl = pl.ds(j * 16, 16)
                        gbuf[b, e, sl] = gbuf[b, e, sl] * wv

        for m in range(M):
            src = x_h.at[m] if shared else x_h.at[m, c]

            def issue(ch, b):
                pltpu.async_copy(src.at[gi_v.at[ch]], gbuf.at[b], sems[b])

            def drain(ch, b):
                pltpu.make_async_copy(
                    src.at[gi_v.at[ch]], gbuf.at[b], sems[b]).wait()

            for k in range(NSLOT - 1):
                issue(k, k)
            pltpu.sync_copy(z_h.at[rows], acc.at[rows])
            plsc.subcore_barrier()

            def swait(b):
                pltpu.make_async_copy(
                    gbuf.at[b], acc.at[si_v.at[0]], ssems[b]).wait()

            @pl.loop(0, QUADS)
            def _quads(j):
                ch0 = j * NSLOT
                for k in range(NSLOT):
                    ch = ch0 + k
                    pre = (k + NSLOT - 1) % NSLOT

                    @pl.when(jnp.logical_and(ch >= 1, ch + NSLOT - 1 < CH))
                    def _sw():
                        swait(pre)

                    @pl.when(ch + NSLOT - 1 < CH)
                    def _pre():
                        issue(ch + NSLOT - 1, pre)

                    drain(ch, k)
                    scale(k, ch)
                    pltpu.async_copy(gbuf.at[k], acc.at[si_v.at[ch]],
                                     ssems[k], add=True)

            for k in range(NSLOT):
                swait(k)
            plsc.subcore_barrier()

            @pl.loop(0, RPT // LANES)
            def _wblk(wb):
                wrows = pl.ds(row0 + wb * LANES, LANES)
                pltpu.sync_copy(acc.at[wrows], gbuf.at[0])

                @pl.loop(0, LANES // 16)
                def _rows(g):
                    rvec = rbuf[pl.ds(wb * LANES + g * 16, 16)]
                    for rr in range(16):
                        rc = rvec[rr]
                        r = g * 16 + rr
                        for j in range(grp):
                            sl = pl.ds(j * 16, 16)
                            gbuf[0, r, sl] = gbuf[0, r, sl] * rc

                pltpu.sync_copy(gbuf.at[0], out_h.at[m, c, wrows])

            if m + 1 < M:
                plsc.subcore_barrier()

    xshape = (M, NP, F) if shared else (M, 2, NP, F)
    return pl.kernel(
        body,
        out_type=_SDS((M, 2, NP, F), jnp.float32),
        mesh=mesh,
        compiler_params=pltpu.CompilerParams(use_tc_tiling_on_sc=False),
        scratch_types=[
            pltpu.VMEM_SHARED((NP, F), jnp.float32),
            pltpu.VMEM((CH, LANES), jnp.int32),
            pltpu.VMEM((CH, LANES), jnp.int32),
            pltpu.VMEM((CH, LANES), jnp.float32),
            pltpu.VMEM((NSLOT, LANES, F), jnp.float32),
            pltpu.VMEM((RPT,), jnp.float32),
            pltpu.SemaphoreType.DMA,
            pltpu.SemaphoreType.DMA,
            pltpu.SemaphoreType.DMA,
            pltpu.SemaphoreType.DMA,
            pltpu.SemaphoreType.DMA,
            pltpu.SemaphoreType.DMA,
            pltpu.SemaphoreType.DMA,
            pltpu.SemaphoreType.DMA,
        ],
    ), xshape


_HOP_D1 = _make_hop(1, FC, True)[0]
_HOP_80S = _make_hop(2, FD, True)[0]
_HOP_80D = _make_hop(2, FD, False)[0]
_HOP_16S = _make_hop(2, FC, True)[0]
_HOP_16D = _make_hop(2, FC, False)[0]
_HOP_64S = _make_hop(2, H, True)[0]
_HOP_64D = _make_hop(2, H, False)[0]


def _recip_body(deg_ref, rc_ref, sfb_ref):
    for c in range(2):
        d = deg_ref[c, :, 0]
        dm = jnp.maximum(d, 1e-8)
        rc_ref[c, :] = 1.0 / dm
        sfb_ref[c, :] = d / dm


def _tc_recip(deg):
    return pl.pallas_call(
        _recip_body,
        out_shape=[_SDS((2, NP), jnp.float32), _SDS((2, NP), jnp.float32)],
    )(deg)


def _stepA_body(h_ref, x_ref, m_ref, wfs_ref, bfs_ref, xs1_ref, hd_ref):
    h = h_ref[0]
    xs1 = h @ wfs_ref[0] + bfs_ref[0, 0]
    mb = m_ref[0] > 0.5
    x1 = jnp.where(mb, x_ref[0], xs1)
    xs1_ref[0] = xs1
    hd_ref[0] = jnp.concatenate(
        [h, x1, m_ref[0], jnp.zeros((RB, FD - H - 2), jnp.float32)], axis=-1)


def _tc_stepA(hst, xt, mt, wfs, bfs):
    return pl.pallas_call(
        _stepA_body,
        grid=(2, NB),
        in_specs=[
            pl.BlockSpec((1, RB, H), lambda m, i: (m, i, 0)),
            pl.BlockSpec((1, RB, 1), lambda m, i: (m, i, 0)),
            pl.BlockSpec((1, RB, 1), lambda m, i: (m, i, 0)),
            pl.BlockSpec((1, H, 1), lambda m, i: (m, 0, 0)),
            pl.BlockSpec((1, 1, 1), lambda m, i: (m, 0, 0)),
        ],
        out_specs=[pl.BlockSpec((1, RB, 1), lambda m, i: (m, i, 0)),
                   pl.BlockSpec((1, RB, FD), lambda m, i: (m, i, 0))],
        out_shape=[_SDS((2, NP, 1), jnp.float32), _SDS((2, NP, FD), jnp.float32)],
    )(hst, xt, mt, wfs, bfs)


def _stepB_body(pf_ref, pb_ref, h_ref, x_ref, m_ref, sfb_ref,
                wb_ref, vf_ref, vb_ref, bo_ref, pa_ref,
                wro_ref, bro_ref,
                xs2_ref, rep_ref, ci_ref):
    h = h_ref[0]
    feats = jnp.concatenate([pf_ref[0, 0], pb_ref[0, 0], h], axis=-1)
    o = (feats @ wb_ref[0] + bo_ref[0, 0]
         + sfb_ref[0, :][:, None] * vf_ref[0, 0]
         + sfb_ref[1, :][:, None] * vb_ref[0, 0])
    a = pa_ref[0, 0, 0]
    o = jnp.where(o >= 0, o, a * o)
    rep = jnp.concatenate([o, h], axis=-1)
    xs2 = rep @ wro_ref[0] + bro_ref[0, 0]
    mb = m_ref[0] > 0.5
    x2 = jnp.where(mb, x_ref[0], xs2)
    xs2_ref[0] = xs2
    rep_ref[0] = rep
    ci_ref[0] = jnp.concatenate(
        [x2, m_ref[0], jnp.zeros((RB, FC - 2), jnp.float32)], axis=-1)


def _tc_stepB(p1, hst, xt, mt, sfb, wB, vf, vb, bo, pa, wro, bro):
    sfb_blk = pl.BlockSpec((2, RB), lambda m, i: (0, i))
    return pl.pallas_call(
        _stepB_body,
        grid=(2, NB),
        in_specs=[
            pl.BlockSpec((1, 1, RB, FD), lambda m, i: (m, 0, i, 0)),
            pl.BlockSpec((1, 1, RB, FD), lambda m, i: (m, 1, i, 0)),
            pl.BlockSpec((1, RB, H), lambda m, i: (m, i, 0)),
            pl.BlockSpec((1, RB, 1), lambda m, i: (m, i, 0)),
            pl.BlockSpec((1, RB, 1), lambda m, i: (m, i, 0)),
            sfb_blk,
            pl.BlockSpec((1, 2 * FD + H, H), lambda m, i: (m, 0, 0)),
            pl.BlockSpec((1, 1, H), lambda m, i: (m, 0, 0)),
            pl.BlockSpec((1, 1, H), lambda m, i: (m, 0, 0)),
            pl.BlockSpec((1, 1, H), lambda m, i: (m, 0, 0)),
            pl.BlockSpec((1, 1, 1), lambda m, i: (m, 0, 0)),
            pl.BlockSpec((1, 2 * H, 1), lambda m, i: (m, 0, 0)),
            pl.BlockSpec((1, 1, 1), lambda m, i: (m, 0, 0)),
        ],
        out_specs=[pl.BlockSpec((1, RB, 1), lambda m, i: (m, i, 0)),
                   pl.BlockSpec((1, RB, 2 * H), lambda m, i: (m, i, 0)),
                   pl.BlockSpec((1, RB, FC), lambda m, i: (m, i, 0))],
        out_shape=[_SDS((2, NP, 1), jnp.float32),
                   _SDS((2, NP, 2 * H), jnp.float32),
                   _SDS((2, NP, FC), jnp.float32)],
    )(p1, p1, hst, xt, mt, sfb, wB, vf, vb, bo, pa, wro, bro)


def _stepC_body(h_ref, hd1f_ref, hd1b_ref, hd2f_ref, hd2b_ref,
                ci_ref, c1f_ref, c1b_ref, c2f_ref, c2b_ref,
                w_ref, b_ref, ru_ref, rh_ref):
    h = h_ref[0]
    feats = jnp.concatenate(
        [h, hd1f_ref[0, 0, :, :H], hd2f_ref[0, 0, :, :H],
         hd1b_ref[0, 0, :, :H], hd2b_ref[0, 0, :, :H],
         ci_ref[0], c1f_ref[0, 0], c2f_ref[0, 0],
         c1b_ref[0, 0], c2b_ref[0, 0]], axis=-1)
    pre = feats @ w_ref[0] + b_ref[0, 0]
    ru = jax.nn.sigmoid(pre)
    ru_ref[0] = ru
    rh_ref[0] = ru[:, :H] * h


def _tc_stepC(hst, p1a, p1b, ci, p2a, p2b, wC, bC):
    kdim = 5 * H + 5 * FC
    bfd = lambda cix: pl.BlockSpec((1, 1, RB, FD), lambda m, i, c=cix: (m, c, i, 0))
    bfc = lambda cix: pl.BlockSpec((1, 1, RB, FC), lambda m, i, c=cix: (m, c, i, 0))
    return pl.pallas_call(
        _stepC_body,
        grid=(2, NB),
        in_specs=[
            pl.BlockSpec((1, RB, H), lambda m, i: (m, i, 0)),
            bfd(0), bfd(1), bfd(0), bfd(1),
            pl.BlockSpec((1, RB, FC), lambda m, i: (m, i, 0)),
            bfc(0), bfc(1), bfc(0), bfc(1),
            pl.BlockSpec((1, kdim, 2 * H), lambda m, i: (m, 0, 0)),
            pl.BlockSpec((1, 1, 2 * H), lambda m, i: (m, 0, 0)),
        ],
        out_specs=[pl.BlockSpec((1, RB, 2 * H), lambda m, i: (m, i, 0)),
                   pl.BlockSpec((1, RB, H), lambda m, i: (m, i, 0))],
        out_shape=[_SDS((2, NP, 2 * H), jnp.float32),
                   _SDS((2, NP, H), jnp.float32)],
    )(hst, p1a, p1a, p1b, p1b, ci, p2a, p2a, p2b, p2b, wC, bC)


def _stepD_body(h_ref, ru_ref, rh_ref, r1f_ref, r1b_ref, r2f_ref, r2b_ref,
                ci_ref, c1f_ref, c1b_ref, c2f_ref, c2b_ref,
                w_ref, b_ref, hn_ref):
    h = h_ref[0]
    feats = jnp.concatenate(
        [rh_ref[0], r1f_ref[0, 0], r2f_ref[0, 0], r1b_ref[0, 0], r2b_ref[0, 0],
         ci_ref[0], c1f_ref[0, 0], c2f_ref[0, 0],
         c1b_ref[0, 0], c2b_ref[0, 0]], axis=-1)
    cc = jnp.tanh(feats @ w_ref[0] + b_ref[0, 0])
    u = ru_ref[0, :, H:]
    hn_ref[0] = u * h + (1.0 - u) * cc


def _tc_stepD(hst, ru, rh, p3a, p3b, ci, p2a, p2b, wD, bD):
    kdim = 5 * H + 5 * FC
    bh = lambda cix: pl.BlockSpec((1, 1, RB, H), lambda m, i, c=cix: (m, c, i, 0))
    bfc = lambda cix: pl.BlockSpec((1, 1, RB, FC), lambda m, i, c=cix: (m, c, i, 0))
    return pl.pallas_call(
        _stepD_body,
        grid=(2, NB),
        in_specs=[
            pl.BlockSpec((1, RB, H), lambda m, i: (m, i, 0)),
            pl.BlockSpec((1, RB, 2 * H), lambda m, i: (m, i, 0)),
            pl.BlockSpec((1, RB, H), lambda m, i: (m, i, 0)),
            bh(0), bh(1), bh(0), bh(1),
            pl.BlockSpec((1, RB, FC), lambda m, i: (m, i, 0)),
            bfc(0), bfc(1), bfc(0), bfc(1),
            pl.BlockSpec((1, kdim, H), lambda m, i: (m, 0, 0)),
            pl.BlockSpec((1, 1, H), lambda m, i: (m, 0, 0)),
        ],
        out_specs=pl.BlockSpec((1, RB, H), lambda m, i: (m, i, 0)),
        out_shape=_SDS((2, NP, H), jnp.float32),
    )(hst, ru, rh, p3a, p3a, p3b, p3b, ci, p2a, p2a, p2b, p2b, wD, bD)


def _read_body(rf_ref, rb_ref, m_ref, e_ref, w1_ref, b1_ref, w2_ref, b2_ref,
               out_ref):
    feats = jnp.concatenate(
        [rf_ref[0, 0], rb_ref[0, 0], m_ref[0], e_ref[...]], axis=-1)
    hid = jnp.maximum(feats @ w1_ref[...] + b1_ref[...], 0.0)
    out_ref[0] = hid @ w2_ref[...] + b2_ref[...]


def _tc_read(rep, mpad, emb, w1, b1, w2, b2):
    kdim = 4 * H + 1 + EMB
    return pl.pallas_call(
        _read_body,
        grid=(S, NB),
        in_specs=[
            pl.BlockSpec((1, 1, RB, 2 * H), lambda t, i: (t, 0, i, 0)),
            pl.BlockSpec((1, 1, RB, 2 * H), lambda t, i: (S - 1 - t, 1, i, 0)),
            pl.BlockSpec((1, RB, 1), lambda t, i: (t, i, 0)),
            pl.BlockSpec((RB, EMB), lambda t, i: (i, 0)),
            pl.BlockSpec((kdim, FF), lambda t, i: (0, 0)),
            pl.BlockSpec((FF,), lambda t, i: (0,)),
            pl.BlockSpec((FF, 128), lambda t, i: (0, 0)),
            pl.BlockSpec((128,), lambda t, i: (0,)),
        ],
        out_specs=pl.BlockSpec((1, RB, 128), lambda t, i: (t, i, 0)),
        out_shape=_SDS((S, NP, 128), jnp.float32),
    )(rep, rep, mpad, emb, w1, b1, w2, b2)


def _prep_model(gp):
    dp = gp['dec']
    wli, bli = dp['lin_in']['W'], dp['lin_in']['b']
    wgf, wgb, bgc = dp['gc']['Wf'], dp['gc']['Wb'], dp['gc']['b']
    wlo, blo = dp['lin_out']['W'], dp['lin_out']['b']
    wz = jnp.concatenate([wli[2:], wli[:2], jnp.zeros((FD - H - 2, H))], 0)
    a_f = wz @ wgf @ wlo[:H]
    a_b = wz @ wgb @ wlo[:H]
    wB = jnp.concatenate([a_f, a_b, wlo[H:]], axis=0)
    vf = bli @ wgf @ wlo[:H]
    vb = bli @ wgb @ wlo[:H]
    bo = bgc @ wlo[:H] + blo

    def conv_w(p, fpart):
        def xpad(w2):
            return jnp.concatenate([w2, jnp.zeros((FC - 2, w2.shape[1]))], 0)
        return jnp.concatenate([
            fpart(p['W0']), fpart(p['Wf'][0]), fpart(p['Wf'][1]),
            fpart(p['Wb'][0]), fpart(p['Wb'][1]),
            xpad(p['W0'][:2]), xpad(p['Wf'][0][:2]), xpad(p['Wf'][1][:2]),
            xpad(p['Wb'][0][:2]), xpad(p['Wb'][1][:2])], axis=0)

    cr, cu, cc = gp['cell']['r'], gp['cell']['u'], gp['cell']['c']
    wC = jnp.concatenate([conv_w(cr, lambda w: w[2:]),
                          conv_w(cu, lambda w: w[2:])], axis=1)
    bC = jnp.concatenate([cr['b'], cu['b']])
    wD = conv_w(cc, lambda w: w[2:])
    bD = cc['b']
    return {
        'wfs': gp['first_stage']['W'],
        'wB': wB, 'vf': vf, 'vb': vb, 'bo': bo,
        'wro': dp['read_out']['W'],
        'wC': wC, 'bC': bC, 'wD': wD, 'bD': bD,
        'h0': gp['h0'],
    }


def _pad_rows(a, np_=NP):
    return jnp.pad(a, ((0, np_ - a.shape[0]),) + ((0, 0),) * (a.ndim - 1))


def kernel(x, edge_index, edge_weight, mask, params):
    f32 = jnp.float32
    src, dst = edge_index[0], edge_index[1]
    padE = EP - E

    def pack(g, s_, w_):
        g = jnp.pad(g, (0, padE)).reshape(TILES, CH, LANES)
        s_ = jnp.pad(s_, (0, padE)).reshape(TILES, CH, LANES)
        w_ = jnp.pad(w_, (0, padE)).reshape(TILES, CH, LANES)
        return g, s_, w_

    g0, s0, w0 = pack(dst, src, edge_weight)
    g1, s1, w1 = pack(src, dst, edge_weight)
    GI = jnp.stack([g0, g1])
    SI = jnp.stack([s0, s1])
    WE = jnp.stack([w0, w1]).astype(f32)

    zeros80 = jnp.zeros((NP, FD), f32)
    zeros64 = jnp.zeros((NP, H), f32)
    zeros16 = jnp.zeros((NP, FC), f32)
    ones_rc = jnp.ones((2, NP), f32)

    deg = _HOP_D1(jnp.ones((1, NP, FC), f32), GI, SI, WE, ones_rc, zeros16)
    recip, sfb = _tc_recip(deg[0])

    pf = _prep_model(params['fwd'])
    pb = _prep_model(params['bwd'])

    def st(k):
        return jnp.stack([pf[k], pb[k]])

    Wfs = st('wfs')
    Bfs = jnp.stack([params['fwd']['first_stage']['b'],
                     params['bwd']['first_stage']['b']])[:, None]
    WB = st('wB')
    Vf, Vb, Bo = st('vf')[:, None], st('vb')[:, None], st('bo')[:, None]
    Pa = jnp.stack([params['fwd']['dec']['prelu_a'],
                    params['bwd']['dec']['prelu_a']])[:, None, None]
    Wro = st('wro')
    Bro = jnp.stack([params['fwd']['dec']['read_out']['b'],
                     params['bwd']['dec']['read_out']['b']])[:, None]
    WC, WD = st('wC'), st('wD')
    BC, BD = st('bC')[:, None], st('bD')[:, None]

    x_pad = jnp.pad(x[0], ((0, 0), (0, NP - N), (0, 0)))
    m_pad = jnp.pad(mask[0], ((0, 0), (0, NP - N), (0, 0)))
    xs_st = jnp.stack([x_pad, x_pad[::-1]], axis=1)
    ms_st = jnp.stack([m_pad, m_pad[::-1]], axis=1)

    hst = jnp.stack([_pad_rows(pf['h0']), _pad_rows(pb['h0'])])

    xs1_l, xs2_l, rep_l = [], [], []
    for t in range(S):
        xt, mt = xs_st[t], ms_st[t]
        xs1, hd = _tc_stepA(hst, xt, mt, Wfs, Bfs)
        p1a = _HOP_80S(hd, GI, SI, WE, recip, zeros80)
        p1b = _HOP_80D(p1a, GI, SI, WE, recip, zeros80)
        xs2, rep, ci = _tc_stepB(p1a, hst, xt, mt, sfb,
                                 WB, Vf, Vb, Bo, Pa, Wro, Bro)
        p2a = _HOP_16S(ci, GI, SI, WE, recip, zeros16)
        p2b = _HOP_16D(p2a, GI, SI, WE, recip, zeros16)
        ru, rh = _tc_stepC(hst, p1a, p1b, ci, p2a, p2b, WC, BC)
        p3a = _HOP_64S(rh, GI, SI, WE, recip, zeros64)
        p3b = _HOP_64D(p3a, GI, SI, WE, recip, zeros64)
        hst = _tc_stepD(hst, ru, rh, p3a, p3b, ci, p2a, p2b, WD, BD)
        xs1_l.append(xs1)
        xs2_l.append(xs2)
        rep_l.append(rep)

    xs1_s = jnp.stack(xs1_l)
    xs2_s = jnp.stack(xs2_l)
    rep_s = jnp.stack(rep_l)

    emb_pad = _pad_rows(params['emb'])
    op = params['out']
    w2_pad = jnp.pad(op['W2'], ((0, 0), (0, 128 - C)))
    b2_pad = jnp.pad(op['b2'], ((0, 128 - C),))
    outr = _tc_read(rep_s, m_pad, emb_pad, op['W1'], op['b1'], w2_pad, b2_pad)

    imputation = outr[:, :N, :C][None]
    fwd_out = xs2_s[:, 0, :N][None]
    bwd_out = xs2_s[::-1, 1, :N][None]
    fwd_pred = xs1_s[:, 0, :N][None]
    bwd_pred = xs1_s[::-1, 1, :N][None]
    return imputation, (fwd_out, bwd_out, fwd_pred, bwd_pred)


# speedup vs baseline: 5.9220x; 1.0328x over previous
import jax
import jax.numpy as jnp
from jax import lax
from jax.experimental import pallas as pl
from jax.experimental.pallas import tpu as pltpu
from jax.experimental.pallas import tpu_sc as plsc

N = 10000
E = 160000
S = 8
C = 1
H = 64
FF = 128
EMB = 16

NP = 10240
TILES = 16
RPT = NP // TILES
LANES = 128
CH = 80
NSLOT = 2
QUADS = CH // NSLOT
EPT = CH * LANES
EP = TILES * EPT
NB = 4
RB = NP // NB
FD = 80
FC = 16

_SDS = jax.ShapeDtypeStruct


def _make_hop(M, F, shared, stage=True):
    mesh = plsc.VectorSubcoreMesh(core_axis_name="c", subcore_axis_name="s")
    grp = F // 16

    def body(x_h, gi_h, si_h, w_h, rc_h, z_h, out_h,
             acc, xs, gi_v, si_v, w_v, gbuf, rbuf, sem0, sem1,
             ssem0, ssem1):
        c = lax.axis_index("c")
        s = lax.axis_index("s")
        row0 = s * RPT
        rows = pl.ds(row0, RPT)
        pltpu.sync_copy(gi_h.at[c, s], gi_v)
        pltpu.sync_copy(si_h.at[c, s], si_v)
        pltpu.sync_copy(w_h.at[c, s], w_v)
        pltpu.sync_copy(rc_h.at[c, rows], rbuf)
        sems = (sem0, sem1)
        ssems = (ssem0, ssem1)

        def scale(b, ch):
            @pl.loop(0, LANES // 16)
            def _edges(g):
                wvec = w_v[ch, pl.ds(g * 16, 16)]
                for ee in range(16):
                    wv = wvec[ee]
                    e = g * 16 + ee
                    for j in range(grp):
                        sl = pl.ds(j * 16, 16)
                        gbuf[b, e, sl] = gbuf[b, e, sl] * wv

        for m in range(M):
            src = x_h.at[m] if shared else x_h.at[m, c]
            gsrc = xs if stage else src

            def issue(ch, b, gsrc=gsrc):
                pltpu.async_copy(gsrc.at[gi_v.at[ch]], gbuf.at[b], sems[b])

            def drain(ch, b, gsrc=gsrc):
                pltpu.make_async_copy(
                    gsrc.at[gi_v.at[ch]], gbuf.at[b], sems[b]).wait()

            if stage:
                pltpu.sync_copy(src.at[rows], xs.at[rows])
            pltpu.sync_copy(z_h.at[rows], acc.at[rows])
            plsc.subcore_barrier()
            for k in range(NSLOT - 1):
                issue(k, k)

            def swait(b):
                pltpu.make_async_copy(
                    gbuf.at[b], acc.at[si_v.at[0]], ssems[b]).wait()

            @pl.loop(0, QUADS)
            def _quads(j):
                ch0 = j * NSLOT
                for k in range(NSLOT):
                    ch = ch0 + k
                    pre = (k + NSLOT - 1) % NSLOT

                    @pl.when(jnp.logical_and(ch >= 1, ch + NSLOT - 1 < CH))
                    def _sw():
                        swait(pre)

                    @pl.when(ch + NSLOT - 1 < CH)
                    def _pre():
                        issue(ch + NSLOT - 1, pre)

                    drain(ch, k)
                    scale(k, ch)
                    pltpu.async_copy(gbuf.at[k], acc.at[si_v.at[ch]],
                                     ssems[k], add=True)

            for k in range(NSLOT):
                swait(k)
            plsc.subcore_barrier()

            @pl.loop(0, RPT // LANES)
            def _wblk(wb):
                wrows = pl.ds(row0 + wb * LANES, LANES)
                pltpu.sync_copy(acc.at[wrows], gbuf.at[0])

                @pl.loop(0, LANES // 16)
                def _rows(g):
                    rvec = rbuf[pl.ds(wb * LANES + g * 16, 16)]
                    for rr in range(16):
                        rc = rvec[rr]
                        r = g * 16 + rr
                        for j in range(grp):
                            sl = pl.ds(j * 16, 16)
                            gbuf[0, r, sl] = gbuf[0, r, sl] * rc

                pltpu.sync_copy(gbuf.at[0], out_h.at[m, c, wrows])

            if m + 1 < M:
                plsc.subcore_barrier()

    xshape = (M, NP, F) if shared else (M, 2, NP, F)
    return pl.kernel(
        body,
        out_type=_SDS((M, 2, NP, F), jnp.float32),
        mesh=mesh,
        compiler_params=pltpu.CompilerParams(use_tc_tiling_on_sc=False),
        scratch_types=[
            pltpu.VMEM_SHARED((NP, F), jnp.float32),
            pltpu.VMEM_SHARED((NP, F) if stage else (16, 16), jnp.float32),
            pltpu.VMEM((CH, LANES), jnp.int32),
            pltpu.VMEM((CH, LANES), jnp.int32),
            pltpu.VMEM((CH, LANES), jnp.float32),
            pltpu.VMEM((NSLOT, LANES, F), jnp.float32),
            pltpu.VMEM((RPT,), jnp.float32),
            pltpu.SemaphoreType.DMA,
            pltpu.SemaphoreType.DMA,
            pltpu.SemaphoreType.DMA,
            pltpu.SemaphoreType.DMA,
        ],
    ), xshape


_HOP_D1 = _make_hop(1, FC, True)[0]
_HOP_80S = _make_hop(2, FD, True, stage=False)[0]
_HOP_80D = _make_hop(2, FD, False, stage=False)[0]
_HOP_16S = _make_hop(2, FC, True)[0]
_HOP_16D = _make_hop(2, FC, False)[0]
_HOP_64S = _make_hop(2, H, True)[0]
_HOP_64D = _make_hop(2, H, False)[0]


def _recip_body(deg_ref, rc_ref, sfb_ref):
    for c in range(2):
        d = deg_ref[c, :, 0]
        dm = jnp.maximum(d, 1e-8)
        rc_ref[c, :] = 1.0 / dm
        sfb_ref[c, :] = d / dm


def _tc_recip(deg):
    return pl.pallas_call(
        _recip_body,
        out_shape=[_SDS((2, NP), jnp.float32), _SDS((2, NP), jnp.float32)],
    )(deg)


def _stepA_body(h_ref, x_ref, m_ref, wfs_ref, bfs_ref, xs1_ref, hd_ref):
    h = h_ref[0]
    xs1 = h @ wfs_ref[0] + bfs_ref[0, 0]
    mb = m_ref[0] > 0.5
    x1 = jnp.where(mb, x_ref[0], xs1)
    xs1_ref[0] = xs1
    hd_ref[0] = jnp.concatenate(
        [h, x1, m_ref[0], jnp.zeros((RB, FD - H - 2), jnp.float32)], axis=-1)


def _tc_stepA(hst, xt, mt, wfs, bfs):
    return pl.pallas_call(
        _stepA_body,
        grid=(2, NB),
        in_specs=[
            pl.BlockSpec((1, RB, H), lambda m, i: (m, i, 0)),
            pl.BlockSpec((1, RB, 1), lambda m, i: (m, i, 0)),
            pl.BlockSpec((1, RB, 1), lambda m, i: (m, i, 0)),
            pl.BlockSpec((1, H, 1), lambda m, i: (m, 0, 0)),
            pl.BlockSpec((1, 1, 1), lambda m, i: (m, 0, 0)),
        ],
        out_specs=[pl.BlockSpec((1, RB, 1), lambda m, i: (m, i, 0)),
                   pl.BlockSpec((1, RB, FD), lambda m, i: (m, i, 0))],
        out_shape=[_SDS((2, NP, 1), jnp.float32), _SDS((2, NP, FD), jnp.float32)],
    )(hst, xt, mt, wfs, bfs)


def _stepB_body(pf_ref, pb_ref, h_ref, x_ref, m_ref, sfb_ref,
                wb_ref, vf_ref, vb_ref, bo_ref, pa_ref,
                wro_ref, bro_ref,
                xs2_ref, rep_ref, ci_ref):
    h = h_ref[0]
    feats = jnp.concatenate([pf_ref[0, 0], pb_ref[0, 0], h], axis=-1)
    o = (feats @ wb_ref[0] + bo_ref[0, 0]
         + sfb_ref[0, :][:, None] * vf_ref[0, 0]
         + sfb_ref[1, :][:, None] * vb_ref[0, 0])
    a = pa_ref[0, 0, 0]
    o = jnp.where(o >= 0, o, a * o)
    rep = jnp.concatenate([o, h], axis=-1)
    xs2 = rep @ wro_ref[0] + bro_ref[0, 0]
    mb = m_ref[0] > 0.5
    x2 = jnp.where(mb, x_ref[0], xs2)
    xs2_ref[0] = xs2
    rep_ref[0] = rep
    ci_ref[0] = jnp.concatenate(
        [x2, m_ref[0], jnp.zeros((RB, FC - 2), jnp.float32)], axis=-1)


def _tc_stepB(p1, hst, xt, mt, sfb, wB, vf, vb, bo, pa, wro, bro):
    sfb_blk = pl.BlockSpec((2, RB), lambda m, i: (0, i))
    return pl.pallas_call(
        _stepB_body,
        grid=(2, NB),
        in_specs=[
            pl.BlockSpec((1, 1, RB, FD), lambda m, i: (m, 0, i, 0)),
            pl.BlockSpec((1, 1, RB, FD), lambda m, i: (m, 1, i, 0)),
            pl.BlockSpec((1, RB, H), lambda m, i: (m, i, 0)),
            pl.BlockSpec((1, RB, 1), lambda m, i: (m, i, 0)),
            pl.BlockSpec((1, RB, 1), lambda m, i: (m, i, 0)),
            sfb_blk,
            pl.BlockSpec((1, 2 * FD + H, H), lambda m, i: (m, 0, 0)),
            pl.BlockSpec((1, 1, H), lambda m, i: (m, 0, 0)),
            pl.BlockSpec((1, 1, H), lambda m, i: (m, 0, 0)),
            pl.BlockSpec((1, 1, H), lambda m, i: (m, 0, 0)),
            pl.BlockSpec((1, 1, 1), lambda m, i: (m, 0, 0)),
            pl.BlockSpec((1, 2 * H, 1), lambda m, i: (m, 0, 0)),
            pl.BlockSpec((1, 1, 1), lambda m, i: (m, 0, 0)),
        ],
        out_specs=[pl.BlockSpec((1, RB, 1), lambda m, i: (m, i, 0)),
                   pl.BlockSpec((1, RB, 2 * H), lambda m, i: (m, i, 0)),
                   pl.BlockSpec((1, RB, FC), lambda m, i: (m, i, 0))],
        out_shape=[_SDS((2, NP, 1), jnp.float32),
                   _SDS((2, NP, 2 * H), jnp.float32),
                   _SDS((2, NP, FC), jnp.float32)],
    )(p1, p1, hst, xt, mt, sfb, wB, vf, vb, bo, pa, wro, bro)


def _stepC_body(h_ref, hd1f_ref, hd1b_ref, hd2f_ref, hd2b_ref,
                ci_ref, c1f_ref, c1b_ref, c2f_ref, c2b_ref,
                w_ref, b_ref, ru_ref, rh_ref):
    h = h_ref[0]
    feats = jnp.concatenate(
        [h, hd1f_ref[0, 0, :, :H], hd2f_ref[0, 0, :, :H],
         hd1b_ref[0, 0, :, :H], hd2b_ref[0, 0, :, :H],
         ci_ref[0], c1f_ref[0, 0], c2f_ref[0, 0],
         c1b_ref[0, 0], c2b_ref[0, 0]], axis=-1)
    pre = feats @ w_ref[0] + b_ref[0, 0]
    ru = jax.nn.sigmoid(pre)
    ru_ref[0] = ru
    rh_ref[0] = ru[:, :H] * h


def _tc_stepC(hst, p1a, p1b, ci, p2a, p2b, wC, bC):
    kdim = 5 * H + 5 * FC
    bfd = lambda cix: pl.BlockSpec((1, 1, RB, FD), lambda m, i, c=cix: (m, c, i, 0))
    bfc = lambda cix: pl.BlockSpec((1, 1, RB, FC), lambda m, i, c=cix: (m, c, i, 0))
    return pl.pallas_call(
        _stepC_body,
        grid=(2, NB),
        in_specs=[
            pl.BlockSpec((1, RB, H), lambda m, i: (m, i, 0)),
            bfd(0), bfd(1), bfd(0), bfd(1),
            pl.BlockSpec((1, RB, FC), lambda m, i: (m, i, 0)),
            bfc(0), bfc(1), bfc(0), bfc(1),
            pl.BlockSpec((1, kdim, 2 * H), lambda m, i: (m, 0, 0)),
            pl.BlockSpec((1, 1, 2 * H), lambda m, i: (m, 0, 0)),
        ],
        out_specs=[pl.BlockSpec((1, RB, 2 * H), lambda m, i: (m, i, 0)),
                   pl.BlockSpec((1, RB, H), lambda m, i: (m, i, 0))],
        out_shape=[_SDS((2, NP, 2 * H), jnp.float32),
                   _SDS((2, NP, H), jnp.float32)],
    )(hst, p1a, p1a, p1b, p1b, ci, p2a, p2a, p2b, p2b, wC, bC)


def _stepD_body(h_ref, ru_ref, rh_ref, r1f_ref, r1b_ref, r2f_ref, r2b_ref,
                ci_ref, c1f_ref, c1b_ref, c2f_ref, c2b_ref,
                w_ref, b_ref, hn_ref):
    h = h_ref[0]
    feats = jnp.concatenate(
        [rh_ref[0], r1f_ref[0, 0], r2f_ref[0, 0], r1b_ref[0, 0], r2b_ref[0, 0],
         ci_ref[0], c1f_ref[0, 0], c2f_ref[0, 0],
         c1b_ref[0, 0], c2b_ref[0, 0]], axis=-1)
    cc = jnp.tanh(feats @ w_ref[0] + b_ref[0, 0])
    u = ru_ref[0, :, H:]
    hn_ref[0] = u * h + (1.0 - u) * cc


def _tc_stepD(hst, ru, rh, p3a, p3b, ci, p2a, p2b, wD, bD):
    kdim = 5 * H + 5 * FC
    bh = lambda cix: pl.BlockSpec((1, 1, RB, H), lambda m, i, c=cix: (m, c, i, 0))
    bfc = lambda cix: pl.BlockSpec((1, 1, RB, FC), lambda m, i, c=cix: (m, c, i, 0))
    return pl.pallas_call(
        _stepD_body,
        grid=(2, NB),
        in_specs=[
            pl.BlockSpec((1, RB, H), lambda m, i: (m, i, 0)),
            pl.BlockSpec((1, RB, 2 * H), lambda m, i: (m, i, 0)),
            pl.BlockSpec((1, RB, H), lambda m, i: (m, i, 0)),
            bh(0), bh(1), bh(0), bh(1),
            pl.BlockSpec((1, RB, FC), lambda m, i: (m, i, 0)),
            bfc(0), bfc(1), bfc(0), bfc(1),
            pl.BlockSpec((1, kdim, H), lambda m, i: (m, 0, 0)),
            pl.BlockSpec((1, 1, H), lambda m, i: (m, 0, 0)),
        ],
        out_specs=pl.BlockSpec((1, RB, H), lambda m, i: (m, i, 0)),
        out_shape=_SDS((2, NP, H), jnp.float32),
    )(hst, ru, rh, p3a, p3a, p3b, p3b, ci, p2a, p2a, p2b, p2b, wD, bD)


def _read_body(rf_ref, rb_ref, m_ref, e_ref, w1_ref, b1_ref, w2_ref, b2_ref,
               out_ref):
    feats = jnp.concatenate(
        [rf_ref[0, 0], rb_ref[0, 0], m_ref[0], e_ref[...]], axis=-1)
    hid = jnp.maximum(feats @ w1_ref[...] + b1_ref[...], 0.0)
    out_ref[0] = hid @ w2_ref[...] + b2_ref[...]


def _tc_read(rep, mpad, emb, w1, b1, w2, b2):
    kdim = 4 * H + 1 + EMB
    return pl.pallas_call(
        _read_body,
        grid=(S, NB),
        in_specs=[
            pl.BlockSpec((1, 1, RB, 2 * H), lambda t, i: (t, 0, i, 0)),
            pl.BlockSpec((1, 1, RB, 2 * H), lambda t, i: (S - 1 - t, 1, i, 0)),
            pl.BlockSpec((1, RB, 1), lambda t, i: (t, i, 0)),
            pl.BlockSpec((RB, EMB), lambda t, i: (i, 0)),
            pl.BlockSpec((kdim, FF), lambda t, i: (0, 0)),
            pl.BlockSpec((FF,), lambda t, i: (0,)),
            pl.BlockSpec((FF, 128), lambda t, i: (0, 0)),
            pl.BlockSpec((128,), lambda t, i: (0,)),
        ],
        out_specs=pl.BlockSpec((1, RB, 128), lambda t, i: (t, i, 0)),
        out_shape=_SDS((S, NP, 128), jnp.float32),
    )(rep, rep, mpad, emb, w1, b1, w2, b2)


def _prep_model(gp):
    dp = gp['dec']
    wli, bli = dp['lin_in']['W'], dp['lin_in']['b']
    wgf, wgb, bgc = dp['gc']['Wf'], dp['gc']['Wb'], dp['gc']['b']
    wlo, blo = dp['lin_out']['W'], dp['lin_out']['b']
    wz = jnp.concatenate([wli[2:], wli[:2], jnp.zeros((FD - H - 2, H))], 0)
    a_f = wz @ wgf @ wlo[:H]
    a_b = wz @ wgb @ wlo[:H]
    wB = jnp.concatenate([a_f, a_b, wlo[H:]], axis=0)
    vf = bli @ wgf @ wlo[:H]
    vb = bli @ wgb @ wlo[:H]
    bo = bgc @ wlo[:H] + blo

    def conv_w(p, fpart):
        def xpad(w2):
            return jnp.concatenate([w2, jnp.zeros((FC - 2, w2.shape[1]))], 0)
        return jnp.concatenate([
            fpart(p['W0']), fpart(p['Wf'][0]), fpart(p['Wf'][1]),
            fpart(p['Wb'][0]), fpart(p['Wb'][1]),
            xpad(p['W0'][:2]), xpad(p['Wf'][0][:2]), xpad(p['Wf'][1][:2]),
            xpad(p['Wb'][0][:2]), xpad(p['Wb'][1][:2])], axis=0)

    cr, cu, cc = gp['cell']['r'], gp['cell']['u'], gp['cell']['c']
    wC = jnp.concatenate([conv_w(cr, lambda w: w[2:]),
                          conv_w(cu, lambda w: w[2:])], axis=1)
    bC = jnp.concatenate([cr['b'], cu['b']])
    wD = conv_w(cc, lambda w: w[2:])
    bD = cc['b']
    return {
        'wfs': gp['first_stage']['W'],
        'wB': wB, 'vf': vf, 'vb': vb, 'bo': bo,
        'wro': dp['read_out']['W'],
        'wC': wC, 'bC': bC, 'wD': wD, 'bD': bD,
        'h0': gp['h0'],
    }


def _pad_rows(a, np_=NP):
    return jnp.pad(a, ((0, np_ - a.shape[0]),) + ((0, 0),) * (a.ndim - 1))


def kernel(x, edge_index, edge_weight, mask, params):
    f32 = jnp.float32
    src, dst = edge_index[0], edge_index[1]
    padE = EP - E

    def pack(g, s_, w_):
        g = jnp.pad(g, (0, padE)).reshape(TILES, CH, LANES)
        s_ = jnp.pad(s_, (0, padE)).reshape(TILES, CH, LANES)
        w_ = jnp.pad(w_, (0, padE)).reshape(TILES, CH, LANES)
        return g, s_, w_

    g0, s0, w0 = pack(dst, src, edge_weight)
    g1, s1, w1 = pack(src, dst, edge_weight)
    GI = jnp.stack([g0, g1])
    SI = jnp.stack([s0, s1])
    WE = jnp.stack([w0, w1]).astype(f32)

    zeros80 = jnp.zeros((NP, FD), f32)
    zeros64 = jnp.zeros((NP, H), f32)
    zeros16 = jnp.zeros((NP, FC), f32)
    ones_rc = jnp.ones((2, NP), f32)

    deg = _HOP_D1(jnp.ones((1, NP, FC), f32), GI, SI, WE, ones_rc, zeros16)
    recip, sfb = _tc_recip(deg[0])

    pf = _prep_model(params['fwd'])
    pb = _prep_model(params['bwd'])

    def st(k):
        return jnp.stack([pf[k], pb[k]])

    Wfs = st('wfs')
    Bfs = jnp.stack([params['fwd']['first_stage']['b'],
                     params['bwd']['first_stage']['b']])[:, None]
    WB = st('wB')
    Vf, Vb, Bo = st('vf')[:, None], st('vb')[:, None], st('bo')[:, None]
    Pa = jnp.stack([params['fwd']['dec']['prelu_a'],
                    params['bwd']['dec']['prelu_a']])[:, None, None]
    Wro = st('wro')
    Bro = jnp.stack([params['fwd']['dec']['read_out']['b'],
                     params['bwd']['dec']['read_out']['b']])[:, None]
    WC, WD = st('wC'), st('wD')
    BC, BD = st('bC')[:, None], st('bD')[:, None]

    x_pad = jnp.pad(x[0], ((0, 0), (0, NP - N), (0, 0)))
    m_pad = jnp.pad(mask[0], ((0, 0), (0, NP - N), (0, 0)))
    xs_st = jnp.stack([x_pad, x_pad[::-1]], axis=1)
    ms_st = jnp.stack([m_pad, m_pad[::-1]], axis=1)

    hst = jnp.stack([_pad_rows(pf['h0']), _pad_rows(pb['h0'])])

    xs1_l, xs2_l, rep_l = [], [], []
    for t in range(S):
        xt, mt = xs_st[t], ms_st[t]
        xs1, hd = _tc_stepA(hst, xt, mt, Wfs, Bfs)
        p1a = _HOP_80S(hd, GI, SI, WE, recip, zeros80)
        p1b = _HOP_80D(p1a, GI, SI, WE, recip, zeros80)
        xs2, rep, ci = _tc_stepB(p1a, hst, xt, mt, sfb,
                                 WB, Vf, Vb, Bo, Pa, Wro, Bro)
        p2a = _HOP_16S(ci, GI, SI, WE, recip, zeros16)
        p2b = _HOP_16D(p2a, GI, SI, WE, recip, zeros16)
        ru, rh = _tc_stepC(hst, p1a, p1b, ci, p2a, p2b, WC, BC)
        p3a = _HOP_64S(rh, GI, SI, WE, recip, zeros64)
        p3b = _HOP_64D(p3a, GI, SI, WE, recip, zeros64)
        hst = _tc_stepD(hst, ru, rh, p3a, p3b, ci, p2a, p2b, WD, BD)
        xs1_l.append(xs1)
        xs2_l.append(xs2)
        rep_l.append(rep)

    xs1_s = jnp.stack(xs1_l)
    xs2_s = jnp.stack(xs2_l)
    rep_s = jnp.stack(rep_l)

    emb_pad = _pad_rows(params['emb'])
    op = params['out']
    w2_pad = jnp.pad(op['W2'], ((0, 0), (0, 128 - C)))
    b2_pad = jnp.pad(op['b2'], ((0, 128 - C),))
    outr = _tc_read(rep_s, m_pad, emb_pad, op['W1'], op['b1'], w2_pad, b2_pad)

    imputation = outr[:, :N, :C][None]
    fwd_out = xs2_s[:, 0, :N][None]
    bwd_out = xs2_s[::-1, 1, :N][None]
    fwd_pred = xs1_s[:, 0, :N][None]
    bwd_pred = xs1_s[::-1, 1, :N][None]
    return imputation, (fwd_out, bwd_out, fwd_pred, bwd_pred)
